# baseline (speedup 1.0000x reference)
import functools

import jax
import jax.numpy as jnp
from jax.experimental import pallas as pl
from jax.experimental.pallas import tpu as pltpu

_K0 = 928981903
_K1 = 3453687069

_B = 128
_V = 100000
_COLS = 512


def _rotl(x, r):
    return (x << jnp.uint32(r)) | (x >> jnp.uint32(32 - r))


def _threefry2x32_zero_hi(x1):
    k0 = jnp.uint32(_K0)
    k1 = jnp.uint32(_K1)
    k2 = k0 ^ k1 ^ jnp.uint32(0x1BD11BDA)
    ks = (k0, k1, k2)
    rot_a = (13, 15, 26, 6)
    rot_b = (17, 29, 16, 24)

    x0 = jnp.broadcast_to(k0, x1.shape)
    x1 = x1 + k1
    for g in range(5):
        for r in (rot_a if g % 2 == 0 else rot_b):
            x0 = x0 + x1
            x1 = _rotl(x1, r) ^ x0
        x0 = x0 + ks[(g + 1) % 3]
        x1 = x1 + ks[(g + 2) % 3] + jnp.uint32(g + 1)
    return x0 ^ x1


def _body(logits_ref, out_ref, best_m, best_i):
    j = pl.program_id(0)
    nblk = pl.num_programs(0)
    col0 = j * _COLS

    shape = (_B, _COLS)
    row = jax.lax.broadcasted_iota(jnp.uint32, shape, 0)
    col = jax.lax.broadcasted_iota(jnp.uint32, shape, 1) + jnp.uint32(col0)

    cnt = row * jnp.uint32(_V) + col
    bits = _threefry2x32_zero_hi(cnt)

    fb = (bits >> jnp.uint32(9)) | jnp.uint32(0x3F800000)
    tiny = jnp.float32(jnp.finfo(jnp.float32).tiny)
    floats = pltpu.bitcast(fb, jnp.float32) - jnp.float32(1.0)
    u = jnp.maximum(tiny, floats * (jnp.float32(1.0) - tiny) + tiny)

    g = -jnp.log(-jnp.log(u))
    phi = logits_ref[...] + g

    valid = col < jnp.uint32(_V)
    neginf = jnp.float32(-jnp.inf)
    phi = jnp.where(valid, phi, neginf)

    bm = jnp.max(phi, axis=1, keepdims=True)
    cidx = col.astype(jnp.int32)
    big = jnp.int32(2**31 - 1)
    bi = jnp.min(jnp.where(phi == bm, cidx, big), axis=1, keepdims=True)

    @pl.when(j == 0)
    def _init():
        best_m[...] = bm
        best_i[...] = bi

    @pl.when(j > 0)
    def _update():
        take = bm > best_m[...]
        best_m[...] = jnp.where(take, bm, best_m[...])
        best_i[...] = jnp.where(take, bi, best_i[...])

    @pl.when(j == nblk - 1)
    def _finish():
        out_ref[...] = best_i[...]


@jax.jit
def kernel(logits):
    nblk = pl.cdiv(_V, _COLS)
    out = pl.pallas_call(
        _body,
        grid=(nblk,),
        in_specs=[pl.BlockSpec((_B, _COLS), lambda j: (0, j))],
        out_specs=pl.BlockSpec((_B, 1), lambda j: (0, 0)),
        out_shape=jax.ShapeDtypeStruct((_B, 1), jnp.int32),
        scratch_shapes=[
            pltpu.VMEM((_B, 1), jnp.float32),
            pltpu.VMEM((_B, 1), jnp.int32),
        ],
    )(logits)
    return out.reshape(_B)


# speedup vs baseline: 1.2009x; 1.2009x over previous
import jax
import jax.numpy as jnp
from jax.experimental import pallas as pl
from jax.experimental.pallas import tpu as pltpu

_K0 = 928981903
_K1 = 3453687069
_KS = (_K0, _K1, _K0 ^ _K1 ^ 0x1BD11BDA)

_B = 128
_V = 100000
_COLS = 512
_ROT = ((13, 15, 26, 6), (17, 29, 16, 24))


def _threefry_bits(x1):
    x0 = None
    for g in range(5):
        for r in _ROT[g & 1]:
            x0 = (x1 + jnp.uint32(_KS[0])) if x0 is None else (x0 + x1)
            x1 = ((x1 << jnp.uint32(r)) | (x1 >> jnp.uint32(32 - r))) ^ x0
        x0 = x0 + jnp.uint32(_KS[(g + 1) % 3])
        x1 = x1 + jnp.uint32((_KS[(g + 2) % 3] + g + 1) & 0xFFFFFFFF)
    return x0 ^ x1


def _phi_of_block(logits, base, j):
    x1 = base + (j * _COLS).astype(jnp.uint32)
    bits = _threefry_bits(x1)
    fb = (bits >> jnp.uint32(9)) | jnp.uint32(0x3F800000)
    tiny = jnp.float32(jnp.finfo(jnp.float32).tiny)
    u = (pltpu.bitcast(fb, jnp.float32) - jnp.float32(1.0)) + tiny
    g = -jnp.log(-jnp.log(u))
    return logits + g


def _body(logits_ref, base_ref, out_ref, runval, runidx):
    j = pl.program_id(0)
    nblk = pl.num_programs(0)

    @pl.when(j == 0)
    def _init():
        runval[...] = jnp.full((_B, 128), -jnp.inf, jnp.float32)
        runidx[...] = jnp.zeros((_B, 128), jnp.int32)

    phi = _phi_of_block(logits_ref[...], base_ref[...], j)
    cidx = jax.lax.broadcasted_iota(jnp.int32, (_B, _COLS), 1) + j * _COLS

    rv = runval[...]
    ri = runidx[...]
    ngrp = _COLS // 128
    for k in range(ngrp):
        p = phi[:, k * 128:(k + 1) * 128]
        ci = cidx[:, k * 128:(k + 1) * 128]
        upd = (p > rv) & (ci < _V)
        rv = jnp.where(upd, p, rv)
        ri = jnp.where(upd, ci, ri)
    runval[...] = rv
    runidx[...] = ri

    @pl.when(j == nblk - 1)
    def _finish():
        rv2 = runval[...]
        ri2 = runidx[...]
        rowmax = jnp.max(rv2, axis=1, keepdims=True)
        big = jnp.int32(2**31 - 1)
        cand = jnp.where(rv2 == rowmax, ri2, big)
        out_ref[...] = jnp.min(cand, axis=1, keepdims=True)


@jax.jit
def kernel(logits):
    nblk = pl.cdiv(_V, _COLS)
    row = jax.lax.broadcasted_iota(jnp.uint32, (_B, _COLS), 0)
    col = jax.lax.broadcasted_iota(jnp.uint32, (_B, _COLS), 1)
    base = row * jnp.uint32(_V) + col + jnp.uint32(_K1)
    out = pl.pallas_call(
        _body,
        grid=(nblk,),
        in_specs=[
            pl.BlockSpec((_B, _COLS), lambda j: (0, j)),
            pl.BlockSpec((_B, _COLS), lambda j: (0, 0)),
        ],
        out_specs=pl.BlockSpec((_B, 1), lambda j: (0, 0)),
        out_shape=jax.ShapeDtypeStruct((_B, 1), jnp.int32),
        scratch_shapes=[
            pltpu.VMEM((_B, 128), jnp.float32),
            pltpu.VMEM((_B, 128), jnp.int32),
        ],
    )(logits, base)
    return out.reshape(_B)
